# baseline (speedup 1.0000x reference)
import functools

import jax
import jax.numpy as jnp
from jax import lax
from jax.experimental import pallas as pl
from jax.experimental.pallas import tpu as pltpu
from jax.experimental.pallas import tpu_sc as plsc

VOCAB = 100000
D_W = 128
D_P = 32
D_OUT = D_W + D_P
B = 1024
S = 200
N = B * S

NC = 2
NS = 16
NW = NC * NS
ROWS_PER_W = N // NW
CHUNK = 128
STEPS = ROWS_PER_W // CHUNK
NB = 4

_mesh = plsc.VectorSubcoreMesh(core_axis_name="c", subcore_axis_name="s")


@functools.partial(
    pl.kernel,
    mesh=_mesh,
    compiler_params=pltpu.CompilerParams(use_tc_tiling_on_sc=False),
    out_type=jax.ShapeDtypeStruct((N, D_OUT), jnp.float32),
    scratch_types=[
        pltpu.VMEM((STEPS, CHUNK), jnp.int32),
        pltpu.VMEM((NB, CHUNK, D_W), jnp.float32),
        pltpu.VMEM((NB, CHUNK, D_P), jnp.float32),
        pltpu.SemaphoreType.DMA((NB,)),
        pltpu.SemaphoreType.DMA((NB,)),
    ],
)
def _emb_kernel(tok_hbm, pos_hbm, w_hbm, out_hbm, idx2, rows, posb, gsem, wsem):
    wid = lax.axis_index("s") * NC + lax.axis_index("c")
    base = wid * ROWS_PER_W

    pltpu.sync_copy(tok_hbm.at[wid], idx2)

    def start_in(g, b):
        pltpu.async_copy(w_hbm.at[idx2.at[g]], rows.at[b], gsem.at[b])
        pltpu.async_copy(pos_hbm.at[pl.ds(base + g * CHUNK, CHUNK)],
                         posb.at[b], gsem.at[b])

    def wait_in(g, b):
        pltpu.make_async_copy(w_hbm.at[idx2.at[g]], rows.at[b],
                              gsem.at[b]).wait()
        pltpu.make_async_copy(pos_hbm.at[pl.ds(base + g * CHUNK, CHUNK)],
                              posb.at[b], gsem.at[b]).wait()

    def start_out(g, b):
        pltpu.async_copy(
            rows.at[b],
            out_hbm.at[pl.ds(base + g * CHUNK, CHUNK), pl.ds(0, D_W)],
            wsem.at[b])
        pltpu.async_copy(
            posb.at[b],
            out_hbm.at[pl.ds(base + g * CHUNK, CHUNK), pl.ds(D_W, D_P)],
            wsem.at[b])

    def wait_out(b):
        pltpu.make_async_copy(
            rows.at[b],
            out_hbm.at[pl.ds(base, CHUNK), pl.ds(0, D_W)],
            wsem.at[b]).wait()
        pltpu.make_async_copy(
            posb.at[b],
            out_hbm.at[pl.ds(base, CHUNK), pl.ds(D_W, D_P)],
            wsem.at[b]).wait()

    start_in(0, 0)
    start_in(1, 1)

    def it(g, carry):
        b = lax.rem(g, NB)
        wait_in(g, b)
        start_out(g, b)
        b2 = lax.rem(g + 2, NB)

        @pl.when(jnp.logical_and(g + 2 < STEPS, g >= NB - 2))
        def _():
            wait_out(b2)

        @pl.when(g + 2 < STEPS)
        def _():
            start_in(g + 2, b2)

        return carry

    lax.fori_loop(0, STEPS, it, 0)
    for b in range(NB):
        wait_out(b)


def kernel(token_ids, pos_onehot, W):
    tok = token_ids.reshape(NW, STEPS, CHUNK).astype(jnp.int32)
    pos = pos_onehot.reshape(N, D_P)
    out = _emb_kernel(tok, pos, W)
    return out.reshape(B, S, D_OUT)


# speedup vs baseline: 1.6183x; 1.6183x over previous
import functools

import jax
import jax.numpy as jnp
from jax import lax
from jax.experimental import pallas as pl
from jax.experimental.pallas import tpu as pltpu
from jax.experimental.pallas import tpu_sc as plsc

VOCAB = 100000
D_W = 128
D_P = 32
D_OUT = D_W + D_P
B = 1024
S = 200
N = B * S

NC = 2
NS = 16
NW = NC * NS
ROWS_PER_W = N // NW
CHUNK = 128
STEPS = ROWS_PER_W // CHUNK
NB = 3

_mesh = plsc.VectorSubcoreMesh(core_axis_name="c", subcore_axis_name="s")


@functools.partial(
    pl.kernel,
    mesh=_mesh,
    out_type=jax.ShapeDtypeStruct((N, D_OUT), jnp.float32),
    scratch_types=[
        pltpu.VMEM((STEPS, CHUNK), jnp.int32),
        pltpu.VMEM((NB, CHUNK, D_W), jnp.float32),
        pltpu.VMEM((NB, CHUNK, D_P), jnp.float32),
        pltpu.SemaphoreType.DMA((NB,)),
        pltpu.SemaphoreType.DMA((NB,)),
    ],
)
def _emb_kernel(tok_hbm, pos_hbm, w_hbm, out_hbm, idx2, rows, posb, gsem, wsem):
    wid = lax.axis_index("s") * NC + lax.axis_index("c")
    base = wid * ROWS_PER_W

    pltpu.sync_copy(tok_hbm.at[wid], idx2)

    def start_in(g, b):
        pltpu.async_copy(w_hbm.at[idx2.at[g]], rows.at[b], gsem.at[b])
        pltpu.async_copy(pos_hbm.at[pl.ds(base + g * CHUNK, CHUNK)],
                         posb.at[b], gsem.at[b])

    def wait_in(g, b):
        pltpu.make_async_copy(w_hbm.at[idx2.at[g]], rows.at[b],
                              gsem.at[b]).wait()
        pltpu.make_async_copy(pos_hbm.at[pl.ds(base + g * CHUNK, CHUNK)],
                              posb.at[b], gsem.at[b]).wait()

    def start_out(g, b):
        pltpu.async_copy(
            rows.at[b],
            out_hbm.at[pl.ds(base + g * CHUNK, CHUNK), pl.ds(0, D_W)],
            wsem.at[b])
        pltpu.async_copy(
            posb.at[b],
            out_hbm.at[pl.ds(base + g * CHUNK, CHUNK), pl.ds(D_W, D_P)],
            wsem.at[b])

    def wait_out(b):
        pltpu.make_async_copy(
            rows.at[b],
            out_hbm.at[pl.ds(base, CHUNK), pl.ds(0, D_W)],
            wsem.at[b]).wait()
        pltpu.make_async_copy(
            posb.at[b],
            out_hbm.at[pl.ds(base, CHUNK), pl.ds(D_W, D_P)],
            wsem.at[b]).wait()

    start_in(0, 0)
    start_in(1, 1)

    def it(g, carry):
        b = lax.rem(g, NB)
        wait_in(g, b)
        start_out(g, b)
        b2 = lax.rem(g + 2, NB)

        @pl.when(jnp.logical_and(g + 2 < STEPS, g >= NB - 2))
        def _():
            wait_out(b2)

        @pl.when(g + 2 < STEPS)
        def _():
            start_in(g + 2, b2)

        return carry

    lax.fori_loop(0, STEPS, it, 0)
    for b in range(NB):
        wait_out(b)


def kernel(token_ids, pos_onehot, W):
    tok = token_ids.reshape(NW, STEPS, CHUNK).astype(jnp.int32)
    pos = pos_onehot.reshape(N, D_P)
    out = _emb_kernel(tok, pos, W)
    return out.reshape(B, S, D_OUT)
